# Initial kernel scaffold; baseline (speedup 1.0000x reference)
import jax
import jax.numpy as jnp
from jax.experimental import pallas as pl


def kernel(atomic_dipoles, charges, positions, batch, ptr):
    raise NotImplementedError("write your pallas kernel here")


# speedup vs baseline: 1.4806x; 1.4806x over previous
import functools

import jax
import jax.numpy as jnp
from jax import lax
from jax.experimental import pallas as pl
from jax.experimental.pallas import tpu as pltpu
from jax.experimental.pallas import tpu_sc as plsc

_FACTOR = 1e-11 / 299792458.0 / 1.602176634e-19
_SCALE = 1.0 / _FACTOR

N_ATOMS = 1600000
N_GRAPHS = 100000


def _build(n_atoms, n_graphs, nc=2, ns=16, ch=1024, sub=128, interpret=False):
    nw = nc * ns
    nsub = ch // sub
    nfull = n_atoms // ch
    tail0 = nfull * ch
    tch = n_atoms - tail0
    assert tch % sub == 0 and tail0 % 8 == 0
    tsub = tch // sub
    maxcpw = -(-nfull // nw)
    tailw = nw - 1
    gpad = -(-n_graphs // (8 * ns)) * 8 * ns
    zr = gpad // ns
    w = 16

    def sc_body(dipf_hbm, posf_hbm, chg_hbm, batchf_hbm, zrows_hbm, out_hbm,
                dipv, posv, chgv, vals, bidx, acc):
        cid = lax.axis_index("c")
        sid = lax.axis_index("s")
        wid = sid * nc + cid

        z0 = pl.multiple_of(sid * zr, 8)
        pltpu.sync_copy(zrows_hbm, acc.at[pl.ds(z0, zr), :])
        plsc.subcore_barrier()

        lane = lax.iota(jnp.int32, 16)
        rowcol = []
        for j in range(3):
            r = (16 * j + lane) // 3
            rowcol.append((r, (16 * j + lane) - 3 * r))

        def do_chunk(a0, nsub_c):
            nrow = nsub_c * sub
            f0 = pl.multiple_of(a0 * 3, 8)
            a0 = pl.multiple_of(a0, 8)
            pltpu.sync_copy(dipf_hbm.at[pl.ds(f0, 3 * nrow)],
                            dipv.at[pl.ds(0, 3 * nrow)])
            pltpu.sync_copy(posf_hbm.at[pl.ds(f0, 3 * nrow)],
                            posv.at[pl.ds(0, 3 * nrow)])
            pltpu.sync_copy(chg_hbm.at[pl.ds(a0, nrow)],
                            chgv.at[pl.ds(0, nrow)])
            for j in range(nsub_c):
                pltpu.sync_copy(
                    batchf_hbm.at[pl.ds(pl.multiple_of(a0 + j * sub, 8), sub)],
                    bidx.at[j])

            def grp(g, carry):
                rbase = g * 16
                fbase = g * 48
                for j in range(3):
                    rj = rbase + rowcol[j][0]
                    d = dipv[pl.ds(fbase + 16 * j, 16)]
                    p = posv[pl.ds(fbase + 16 * j, 16)]
                    q = plsc.load_gather(chgv, [rj])
                    plsc.store_scatter(vals, [rj, rowcol[j][1]],
                                       d + p * q * _SCALE)
                return carry

            lax.fori_loop(0, nrow // 16, grp, 0)

            for j in range(nsub_c):
                pltpu.sync_copy(vals.at[pl.ds(j * sub, sub), :],
                                acc.at[bidx.at[j]], add=True)

        def chunk_iter(ci, carry):
            cg = wid + nw * ci

            @pl.when(cg < nfull)
            def _():
                do_chunk(cg * ch, nsub)

            return carry

        lax.fori_loop(0, maxcpw, chunk_iter, 0)

        if tsub:
            @pl.when(wid == tailw)
            def _():
                do_chunk(tail0, tsub)

        plsc.subcore_barrier()
        pltpu.sync_copy(acc.at[pl.ds(z0, zr), :],
                        out_hbm.at[cid, pl.ds(z0, zr), :])

    sc_kernel = functools.partial(
        pl.kernel,
        compiler_params=pltpu.CompilerParams(
            needs_layout_passes=False, use_tc_tiling_on_sc=False),
        out_type=jax.ShapeDtypeStruct((nc, gpad, w), jnp.float32),
        mesh=plsc.VectorSubcoreMesh(
            core_axis_name="c", subcore_axis_name="s",
            num_cores=nc, num_subcores=ns),
        scratch_types=[
            pltpu.VMEM((ch * 3,), jnp.float32),
            pltpu.VMEM((ch * 3,), jnp.float32),
            pltpu.VMEM((ch,), jnp.float32),
            pltpu.VMEM((ch, w), jnp.float32),
            pltpu.VMEM((nsub, sub), jnp.int32),
            pltpu.VMEM_SHARED((gpad, w), jnp.float32),
        ],
        interpret=interpret,
    )(sc_body)

    def combine_body(p_ref, o_ref):
        o_ref[...] = p_ref[0] + p_ref[1]

    def run(atomic_dipoles, charges, positions, batch, ptr):
        del ptr
        dipf = atomic_dipoles.reshape(n_atoms * 3)
        posf = positions.reshape(n_atoms * 3)
        batchf = batch.astype(jnp.int32)
        zrows = jnp.zeros((zr, w), jnp.float32)
        partials = sc_kernel(dipf, posf, charges, batchf, zrows)
        summed = pl.pallas_call(
            combine_body,
            out_shape=jax.ShapeDtypeStruct((gpad * w // 128, 128),
                                           jnp.float32),
            interpret=interpret,
        )(partials.reshape(nc, gpad * w // 128, 128))
        return summed.reshape(gpad, w)[:n_graphs, :3]

    return run


_run = _build(N_ATOMS, N_GRAPHS)


def kernel(atomic_dipoles, charges, positions, batch, ptr):
    return _run(atomic_dipoles, charges, positions, batch, ptr)


# speedup vs baseline: 15.8768x; 10.7231x over previous
import functools

import jax
import jax.numpy as jnp
from jax import lax
from jax.experimental import pallas as pl
from jax.experimental.pallas import tpu as pltpu
from jax.experimental.pallas import tpu_sc as plsc

_FACTOR = 1e-11 / 299792458.0 / 1.602176634e-19
_SCALE = 1.0 / _FACTOR

N_ATOMS = 1600000
N_GRAPHS = 100000


def _build(n_atoms, n_graphs, nc=2, ns=16, ch=1024, sub=128, interpret=False):
    nw = nc * ns
    nsub = ch // sub
    nfull = n_atoms // ch
    tail0 = nfull * ch
    tch = n_atoms - tail0
    assert tch % sub == 0 and tail0 % 8 == 0
    tsub = tch // sub
    maxcpw = -(-nfull // nw)
    tailw = nw - 1
    gpad = -(-n_graphs // (8 * ns)) * 8 * ns
    zr = gpad // ns
    w = 16

    def sc_body(dipt_hbm, post_hbm, chg_hbm, batchf_hbm, zrows_hbm, out_hbm,
                dipv, posv, chgv, vals, bidx, acc, sem_in, sem_sc):
        cid = lax.axis_index("c")
        sid = lax.axis_index("s")
        wid = sid * nc + cid

        z0 = pl.multiple_of(sid * zr, 8)
        pltpu.sync_copy(zrows_hbm, acc.at[pl.ds(z0, zr), :])
        plsc.subcore_barrier()

        lane = lax.iota(jnp.int32, 16)

        def fire_inputs(a0, nrow, bsel):
            a0 = pl.multiple_of(a0, sub)
            cps = [
                pltpu.async_copy(dipt_hbm.at[:, pl.ds(a0, nrow)],
                                 dipv.at[:, pl.ds(0, nrow)], sem_in),
                pltpu.async_copy(post_hbm.at[:, pl.ds(a0, nrow)],
                                 posv.at[:, pl.ds(0, nrow)], sem_in),
                pltpu.async_copy(chg_hbm.at[pl.ds(a0, nrow)],
                                 chgv.at[pl.ds(0, nrow)], sem_in),
            ]
            for j in range(nrow // sub):
                cps.append(pltpu.async_copy(
                    batchf_hbm.at[pl.ds(pl.multiple_of(a0 + j * sub, 8), sub)],
                    bidx.at[bsel, j], sem_in))
            return cps

        def drain(cps):
            for c in cps:
                c.wait()

        def fire_scatters(nsub_c, bsel):
            for j in range(nsub_c):
                pltpu.async_copy(vals.at[pl.ds(j * sub, sub), :],
                                 acc.at[bidx.at[bsel, j]], sem_sc, add=True)

        def drain_scatters(nsub_c):
            for j in range(nsub_c):
                pltpu.make_async_copy(vals.at[pl.ds(j * sub, sub), :],
                                      acc.at[bidx.at[0, j]], sem_sc).wait()

        colc = [lane * 0 + c for c in range(3)]

        def compute(nrow):
            def grp(k, carry):
                rbase = k * 16
                rows = rbase + lane
                for c in range(3):
                    v = (dipv[c, pl.ds(rbase, 16)]
                         + posv[c, pl.ds(rbase, 16)]
                         * chgv[pl.ds(rbase, 16)] * _SCALE)
                    plsc.store_scatter(vals, [rows, colc[c]], v)
                return carry

            lax.fori_loop(0, nrow // 16, grp, 0)

        def chunk_iter(ci, carry):
            cg = wid + nw * ci

            @pl.when(cg < nfull)
            def _():
                cps = fire_inputs(cg * ch, ch, ci % 2)

                @pl.when(ci > 0)
                def _():
                    drain_scatters(nsub)

                drain(cps)
                compute(ch)
                fire_scatters(nsub, ci % 2)

            return carry

        lax.fori_loop(0, maxcpw, chunk_iter, 0)
        drain_scatters(nsub)

        if tsub:
            @pl.when(wid == tailw)
            def _():
                cps = fire_inputs(tail0, tch, 0)
                drain(cps)
                compute(tch)
                fire_scatters(tsub, 0)
                drain_scatters(tsub)

        plsc.subcore_barrier()
        pltpu.sync_copy(acc.at[pl.ds(z0, zr), :],
                        out_hbm.at[cid, pl.ds(z0, zr), :])

    sc_kernel = functools.partial(
        pl.kernel,
        compiler_params=pltpu.CompilerParams(
            needs_layout_passes=False, use_tc_tiling_on_sc=False),
        out_type=jax.ShapeDtypeStruct((nc, gpad, w), jnp.float32),
        mesh=plsc.VectorSubcoreMesh(
            core_axis_name="c", subcore_axis_name="s",
            num_cores=nc, num_subcores=ns),
        scratch_types=[
            pltpu.VMEM((3, ch), jnp.float32),
            pltpu.VMEM((3, ch), jnp.float32),
            pltpu.VMEM((ch,), jnp.float32),
            pltpu.VMEM((ch, w), jnp.float32),
            pltpu.VMEM((2, nsub, sub), jnp.int32),
            pltpu.VMEM_SHARED((gpad, w), jnp.float32),
            pltpu.SemaphoreType.DMA,
            pltpu.SemaphoreType.DMA,
        ],
        interpret=interpret,
    )(sc_body)

    def combine_body(p_ref, o_ref):
        o_ref[...] = p_ref[0] + p_ref[1]

    def run(atomic_dipoles, charges, positions, batch, ptr):
        del ptr
        dipt = atomic_dipoles.T
        post = positions.T
        batchf = batch.astype(jnp.int32)
        zrows = jnp.zeros((zr, w), jnp.float32)
        partials = sc_kernel(dipt, post, charges, batchf, zrows)
        summed = pl.pallas_call(
            combine_body,
            out_shape=jax.ShapeDtypeStruct((gpad * w // 128, 128),
                                           jnp.float32),
            interpret=interpret,
        )(partials.reshape(nc, gpad * w // 128, 128))
        return summed.reshape(gpad, w)[:n_graphs, :3]

    return run


_run = _build(N_ATOMS, N_GRAPHS)


def kernel(atomic_dipoles, charges, positions, batch, ptr):
    return _run(atomic_dipoles, charges, positions, batch, ptr)


# speedup vs baseline: 37.6346x; 2.3704x over previous
import functools

import jax
import jax.numpy as jnp
from jax import lax
from jax.experimental import pallas as pl
from jax.experimental.pallas import tpu as pltpu
from jax.experimental.pallas import tpu_sc as plsc

_FACTOR = 1e-11 / 299792458.0 / 1.602176634e-19
_SCALE = 1.0 / _FACTOR

N_ATOMS = 1600000
N_GRAPHS = 100000


def _build(n_atoms, n_graphs, nc=2, ns=16, ch=1024, sub=128, interpret=False):
    nw = nc * ns
    nsub = ch // sub
    nfull = n_atoms // ch
    tail0 = nfull * ch
    tch = n_atoms - tail0
    assert tch % sub == 0 and tail0 % 8 == 0
    tsub = tch // sub
    maxcpw = -(-nfull // nw)
    tailw = nw - 1
    gpad = -(-n_graphs // (8 * ns)) * 8 * ns
    zr = gpad // ns
    w = 16

    def sc_body(dx_hbm, dy_hbm, dz_hbm, px_hbm, py_hbm, pz_hbm,
                chg_hbm, batchf_hbm, zrows_hbm, out_hbm,
                dipv, posv, chgv, vals, bidx, acc, sem_in, sem_sc):
        cid = lax.axis_index("c")
        sid = lax.axis_index("s")
        wid = sid * nc + cid

        z0 = pl.multiple_of(sid * zr, 8)
        pltpu.sync_copy(zrows_hbm, acc.at[pl.ds(z0, zr), :])
        plsc.subcore_barrier()

        lane = lax.iota(jnp.int32, 16)

        def fire_inputs(a0, nrow, bsel):
            a0 = pl.multiple_of(a0, sub)
            cps = [
                pltpu.async_copy(src.at[pl.ds(a0, nrow)],
                                 dst.at[c, pl.ds(0, nrow)], sem_in)
                for src, dst, c in
                [(dx_hbm, dipv, 0), (dy_hbm, dipv, 1), (dz_hbm, dipv, 2),
                 (px_hbm, posv, 0), (py_hbm, posv, 1), (pz_hbm, posv, 2)]
            ]
            cps.append(pltpu.async_copy(chg_hbm.at[pl.ds(a0, nrow)],
                                        chgv.at[pl.ds(0, nrow)], sem_in))
            for j in range(nrow // sub):
                cps.append(pltpu.async_copy(
                    batchf_hbm.at[pl.ds(pl.multiple_of(a0 + j * sub, 8), sub)],
                    bidx.at[bsel, j], sem_in))
            return cps

        def drain(cps):
            for c in cps:
                c.wait()

        def fire_scatters(nsub_c, bsel):
            for j in range(nsub_c):
                pltpu.async_copy(vals.at[pl.ds(j * sub, sub), :],
                                 acc.at[bidx.at[bsel, j]], sem_sc, add=True)

        def drain_scatters(nsub_c):
            for j in range(nsub_c):
                pltpu.make_async_copy(vals.at[pl.ds(j * sub, sub), :],
                                      acc.at[bidx.at[0, j]], sem_sc).wait()

        colc = [lane * 0 + c for c in range(3)]

        def compute(nrow):
            def grp(k, carry):
                rbase = k * 16
                rows = rbase + lane
                for c in range(3):
                    v = (dipv[c, pl.ds(rbase, 16)]
                         + posv[c, pl.ds(rbase, 16)]
                         * chgv[pl.ds(rbase, 16)] * _SCALE)
                    plsc.store_scatter(vals, [rows, colc[c]], v)
                return carry

            lax.fori_loop(0, nrow // 16, grp, 0)

        def chunk_iter(ci, carry):
            cg = wid + nw * ci

            @pl.when(cg < nfull)
            def _():
                cps = fire_inputs(cg * ch, ch, ci % 2)

                @pl.when(ci > 0)
                def _():
                    drain_scatters(nsub)

                drain(cps)
                compute(ch)
                fire_scatters(nsub, ci % 2)

            return carry

        lax.fori_loop(0, maxcpw, chunk_iter, 0)
        drain_scatters(nsub)

        if tsub:
            @pl.when(wid == tailw)
            def _():
                cps = fire_inputs(tail0, tch, 0)
                drain(cps)
                compute(tch)
                fire_scatters(tsub, 0)
                drain_scatters(tsub)

        plsc.subcore_barrier()
        pltpu.sync_copy(acc.at[pl.ds(z0, zr), :],
                        out_hbm.at[cid, pl.ds(z0, zr), :])

    sc_kernel = functools.partial(
        pl.kernel,
        compiler_params=pltpu.CompilerParams(
            needs_layout_passes=False, use_tc_tiling_on_sc=False),
        out_type=jax.ShapeDtypeStruct((nc, gpad, w), jnp.float32),
        mesh=plsc.VectorSubcoreMesh(
            core_axis_name="c", subcore_axis_name="s",
            num_cores=nc, num_subcores=ns),
        scratch_types=[
            pltpu.VMEM((3, ch), jnp.float32),
            pltpu.VMEM((3, ch), jnp.float32),
            pltpu.VMEM((ch,), jnp.float32),
            pltpu.VMEM((ch, w), jnp.float32),
            pltpu.VMEM((2, nsub, sub), jnp.int32),
            pltpu.VMEM_SHARED((gpad, w), jnp.float32),
            pltpu.SemaphoreType.DMA,
            pltpu.SemaphoreType.DMA,
        ],
        interpret=interpret,
    )(sc_body)

    def combine_body(p_ref, o_ref):
        o_ref[...] = p_ref[0] + p_ref[1]

    def run(atomic_dipoles, charges, positions, batch, ptr):
        del ptr
        batchf = batch.astype(jnp.int32)
        zrows = jnp.zeros((zr, w), jnp.float32)
        partials = sc_kernel(
            atomic_dipoles[:, 0], atomic_dipoles[:, 1], atomic_dipoles[:, 2],
            positions[:, 0], positions[:, 1], positions[:, 2],
            charges, batchf, zrows)
        summed = pl.pallas_call(
            combine_body,
            out_shape=jax.ShapeDtypeStruct((gpad * w // 128, 128),
                                           jnp.float32),
            interpret=interpret,
        )(partials.reshape(nc, gpad * w // 128, 128))
        return summed.reshape(gpad, w)[:n_graphs, :3]

    return run


_run = _build(N_ATOMS, N_GRAPHS)


def kernel(atomic_dipoles, charges, positions, batch, ptr):
    return _run(atomic_dipoles, charges, positions, batch, ptr)


# speedup vs baseline: 37.7168x; 1.0022x over previous
import functools

import jax
import jax.numpy as jnp
from jax import lax
from jax.experimental import pallas as pl
from jax.experimental.pallas import tpu as pltpu
from jax.experimental.pallas import tpu_sc as plsc

_FACTOR = 1e-11 / 299792458.0 / 1.602176634e-19
_SCALE = 1.0 / _FACTOR

N_ATOMS = 1600000
N_GRAPHS = 100000


def _build(n_atoms, n_graphs, nc=2, ns=16, ch=1024, sub=128, interpret=False):
    nw = nc * ns
    nsub = ch // sub
    nfull = n_atoms // ch
    tail0 = nfull * ch
    tch = n_atoms - tail0
    assert tch % sub == 0 and tail0 % 8 == 0
    tsub = tch // sub
    maxcpw = -(-nfull // nw)
    tailw = nw - 1
    gpad = -(-n_graphs // (8 * ns)) * 8 * ns
    zr = gpad // ns
    w = 16

    def sc_body(dx_hbm, dy_hbm, dz_hbm, px_hbm, py_hbm, pz_hbm,
                chg_hbm, batchf_hbm, zrows_hbm, out_hbm,
                dipv, posv, chgv, vals, bidx, acc, sem_in, sem_sc):
        cid = lax.axis_index("c")
        sid = lax.axis_index("s")
        wid = sid * nc + cid

        z0 = pl.multiple_of(sid * zr, 8)
        pltpu.sync_copy(zrows_hbm, acc.at[pl.ds(z0, zr), :])
        plsc.subcore_barrier()

        lane = lax.iota(jnp.int32, 16)

        def fire_inputs(a0, nrow, bsel):
            a0 = pl.multiple_of(a0, sub)
            cps = [
                pltpu.async_copy(src.at[pl.ds(a0, nrow)],
                                 dst.at[c, pl.ds(0, nrow)], sem_in)
                for src, dst, c in
                [(dx_hbm, dipv, 0), (dy_hbm, dipv, 1), (dz_hbm, dipv, 2),
                 (px_hbm, posv, 0), (py_hbm, posv, 1), (pz_hbm, posv, 2)]
            ]
            cps.append(pltpu.async_copy(chg_hbm.at[pl.ds(a0, nrow)],
                                        chgv.at[pl.ds(0, nrow)], sem_in))
            for j in range(nrow // sub):
                cps.append(pltpu.async_copy(
                    batchf_hbm.at[pl.ds(pl.multiple_of(a0 + j * sub, 8), sub)],
                    bidx.at[bsel, j], sem_in))
            return cps

        def drain(cps):
            for c in cps:
                c.wait()

        def fire_scatters(nsub_c, bsel):
            for j in range(nsub_c):
                pltpu.async_copy(vals.at[pl.ds(j * sub, sub), :],
                                 acc.at[bidx.at[bsel, j]], sem_sc, add=True)

        def drain_scatters(nsub_c):
            for j in range(nsub_c):
                pltpu.make_async_copy(vals.at[pl.ds(j * sub, sub), :],
                                      acc.at[bidx.at[0, j]], sem_sc).wait()

        colc = [lane * 0 + c for c in range(3)]

        def compute(nrow):
            unroll = 4

            def grp(k, carry):
                for u in range(unroll):
                    rbase = k * (16 * unroll) + u * 16
                    rows = rbase + lane
                    for c in range(3):
                        v = (dipv[c, pl.ds(rbase, 16)]
                             + posv[c, pl.ds(rbase, 16)]
                             * chgv[pl.ds(rbase, 16)] * _SCALE)
                        plsc.store_scatter(vals, [rows, colc[c]], v)
                return carry

            lax.fori_loop(0, nrow // (16 * unroll), grp, 0)

        def chunk_iter(ci, carry):
            cg = wid + nw * ci

            @pl.when(cg < nfull)
            def _():
                cps = fire_inputs(cg * ch, ch, ci % 2)

                @pl.when(ci > 0)
                def _():
                    drain_scatters(nsub)

                drain(cps)
                compute(ch)
                fire_scatters(nsub, ci % 2)

            return carry

        lax.fori_loop(0, maxcpw, chunk_iter, 0)
        drain_scatters(nsub)

        if tsub:
            @pl.when(wid == tailw)
            def _():
                cps = fire_inputs(tail0, tch, 0)
                drain(cps)
                compute(tch)
                fire_scatters(tsub, 0)
                drain_scatters(tsub)

        plsc.subcore_barrier()
        pltpu.sync_copy(acc.at[pl.ds(z0, zr), :],
                        out_hbm.at[cid, pl.ds(z0, zr), :])

    sc_kernel = functools.partial(
        pl.kernel,
        compiler_params=pltpu.CompilerParams(
            needs_layout_passes=False, use_tc_tiling_on_sc=False),
        out_type=jax.ShapeDtypeStruct((nc, gpad, w), jnp.float32),
        mesh=plsc.VectorSubcoreMesh(
            core_axis_name="c", subcore_axis_name="s",
            num_cores=nc, num_subcores=ns),
        scratch_types=[
            pltpu.VMEM((3, ch), jnp.float32),
            pltpu.VMEM((3, ch), jnp.float32),
            pltpu.VMEM((ch,), jnp.float32),
            pltpu.VMEM((ch, w), jnp.float32),
            pltpu.VMEM((2, nsub, sub), jnp.int32),
            pltpu.VMEM_SHARED((gpad, w), jnp.float32),
            pltpu.SemaphoreType.DMA,
            pltpu.SemaphoreType.DMA,
        ],
        interpret=interpret,
    )(sc_body)

    def combine_body(p_ref, o_ref):
        o_ref[...] = p_ref[0] + p_ref[1]

    def run(atomic_dipoles, charges, positions, batch, ptr):
        del ptr
        batchf = batch.astype(jnp.int32)
        zrows = jnp.zeros((zr, w), jnp.float32)
        partials = sc_kernel(
            atomic_dipoles[:, 0], atomic_dipoles[:, 1], atomic_dipoles[:, 2],
            positions[:, 0], positions[:, 1], positions[:, 2],
            charges, batchf, zrows)
        summed = pl.pallas_call(
            combine_body,
            out_shape=jax.ShapeDtypeStruct((gpad * w // 128, 128),
                                           jnp.float32),
            interpret=interpret,
        )(partials.reshape(nc, gpad * w // 128, 128))
        return summed.reshape(gpad, w)[:n_graphs, :3]

    return run


_run = _build(N_ATOMS, N_GRAPHS)


def kernel(atomic_dipoles, charges, positions, batch, ptr):
    return _run(atomic_dipoles, charges, positions, batch, ptr)
